# baseline (speedup 1.0000x reference)
import jax
import jax.numpy as jnp
from jax.experimental import pallas as pl

NN = 128


def _sqdist(A, B):
    a2 = jnp.sum(A * A, axis=-1, keepdims=True)
    b2 = jnp.sum(B * B, axis=-1, keepdims=True)
    d2 = a2 + jnp.swapaxes(b2, -1, -2) - 2.0 * jnp.einsum('...md,...nd->...mn', A, B)
    return jnp.clip(d2, 0.0, None)


def _rbf(A, B, l, a):
    return a * jnp.exp(-_sqdist(A, B) / (2.0 * l ** 2))


def _copy_kernel(x_ref, o_ref):
    o_ref[...] = x_ref[...]


def kernel(x, trainX, trainy, l, a):
    d2 = _sqdist(x, trainX)
    _, neighbors = jax.lax.top_k(-d2, NN + 1)
    idx = neighbors[:, 1:]
    nX = trainX[idx]
    ny = trainy[idx]
    auto = _rbf(nX, nX, l, a)
    autoCov = jnp.linalg.inv(auto)
    crossCov = _rbf(x[:, None, :], nX, l, a)
    kWeights = crossCov @ autoCov
    y = (kWeights @ ny)[:, 0, 0]
    yVar = a * jnp.ones(x.shape[0], dtype=x.dtype) - jnp.squeeze(
        kWeights @ jnp.swapaxes(crossCov, 1, 2))
    y = pl.pallas_call(
        _copy_kernel,
        out_shape=jax.ShapeDtypeStruct(y.shape, y.dtype),
    )(y)
    return (y, yVar)


# speedup vs baseline: 1.8448x; 1.8448x over previous
import functools

import jax
import jax.numpy as jnp
from jax import lax
from jax.experimental import pallas as pl
from jax.experimental.pallas import tpu as pltpu
from jax.experimental.pallas import tpu_sc as plsc

NN = 128
K = NN + 1
N = 100000
TB = 2048
NPAD = 100352
NBLK = NPAD // TB
QT = 128
NQT = 1024 // QT
CMAX = 384
PADV = 1e30


def _k1_body(x_ref, t_ref, a2_ref, b2_ref, d2_ref, m_ref, mx_ref):
    j = pl.program_id(1)
    xb = x_ref[...]
    tb = t_ref[...]
    dot = lax.dot_general(xb, tb, (((1,), (1,)), ((), ())),
                          preferred_element_type=jnp.float32)
    s = a2_ref[...] + b2_ref[0]
    d2 = jnp.maximum(s - 2.0 * dot, 0.0)
    colid = j * TB + lax.broadcasted_iota(jnp.int32, (QT, TB), 1)
    pad = colid >= N
    d2w = jnp.where(pad, PADV, d2)
    d2_ref[...] = d2w

    @pl.when(j == 0)
    def _():
        m_ref[...] = jnp.full((QT, 1), PADV, jnp.float32)
        mx_ref[...] = jnp.full((QT, 1), -PADV, jnp.float32)

    m_ref[...] = jnp.minimum(m_ref[...], jnp.min(d2w, axis=1, keepdims=True))
    mx_ref[...] = jnp.maximum(
        mx_ref[...], jnp.max(jnp.where(pad, -PADV, d2), axis=1, keepdims=True))


def _k1(x, tXp, a2, b2r):
    return pl.pallas_call(
        _k1_body,
        grid=(NQT, NBLK),
        in_specs=[
            pl.BlockSpec((QT, 32), lambda q, j: (q, 0)),
            pl.BlockSpec((TB, 32), lambda q, j: (j, 0)),
            pl.BlockSpec((QT, 1), lambda q, j: (q, 0)),
            pl.BlockSpec((1, 1, TB), lambda q, j: (j, 0, 0)),
        ],
        out_specs=[
            pl.BlockSpec((QT, TB), lambda q, j: (q, j)),
            pl.BlockSpec((QT, 1), lambda q, j: (q, 0)),
            pl.BlockSpec((QT, 1), lambda q, j: (q, 0)),
        ],
        out_shape=[
            jax.ShapeDtypeStruct((1024, NPAD), jnp.float32),
            jax.ShapeDtypeStruct((1024, 1), jnp.float32),
            jax.ShapeDtypeStruct((1024, 1), jnp.float32),
        ],
    )(x, tXp, a2, b2r)


def _k2_body(d2_ref, t_ref, cnt_ref):
    j = pl.program_id(1)

    @pl.when(j == 0)
    def _():
        cnt_ref[...] = jnp.zeros((QT, 4), jnp.float32)

    d2b = d2_ref[...]
    t = t_ref[...]
    cols = [jnp.sum((d2b < t[:, kk:kk + 1]).astype(jnp.float32), axis=1)
            for kk in range(4)]
    cnt_ref[...] += jnp.stack(cols, axis=1)


def _k2(d2, probes):
    return pl.pallas_call(
        _k2_body,
        grid=(NQT, NBLK),
        in_specs=[
            pl.BlockSpec((QT, TB), lambda q, j: (q, j)),
            pl.BlockSpec((QT, 4), lambda q, j: (q, 0)),
        ],
        out_specs=pl.BlockSpec((QT, 4), lambda q, j: (q, 0)),
        out_shape=jax.ShapeDtypeStruct((1024, 4), jnp.float32),
    )(d2, probes)


def _find_thresholds(d2, m, mx):
    lo0 = m[:, 0]
    hi0 = mx[:, 0] * 1.000001 + 1.0
    cnt0 = jnp.full((1024,), float(N), jnp.float32)

    def cond(state):
        it, lo, hi, cnt = state
        return jnp.logical_and(it < 40, jnp.any(cnt > CMAX))

    def body(state):
        it, lo, hi, cnt = state
        w = hi - lo
        probes = jnp.stack([lo + w * (kk / 5.0) for kk in (1.0, 2.0, 3.0, 4.0)],
                           axis=1)
        c = _k2(d2, probes)
        bs = jnp.concatenate([lo[:, None], probes, hi[:, None]], axis=1)
        ns = jnp.concatenate(
            [jnp.zeros((1024, 1), jnp.float32), c, cnt[:, None]], axis=1)
        sel = jnp.sum((c < float(K)).astype(jnp.int32), axis=1)
        lo2 = jnp.take_along_axis(bs, sel[:, None], 1)[:, 0]
        hi2 = jnp.take_along_axis(bs, sel[:, None] + 1, 1)[:, 0]
        cnt2 = jnp.take_along_axis(ns, sel[:, None] + 1, 1)[:, 0]
        return it + 1, lo2, hi2, cnt2

    _, _, hi, _ = lax.while_loop(cond, body, (jnp.int32(0), lo0, hi0, cnt0))
    return hi


def _compact_sc(d2, thresh):
    info = plsc.get_sparse_core_info()
    nc, ns, L = info.num_cores, info.num_subcores, info.num_lanes
    nw = nc * ns
    qw = 1024 // nw
    mesh = plsc.VectorSubcoreMesh(core_axis_name="c", subcore_axis_name="s")
    threshb = jnp.broadcast_to(thresh[:, None], (1024, L))

    @functools.partial(
        pl.kernel, mesh=mesh,
        out_type=[jax.ShapeDtypeStruct((1024, CMAX), jnp.int32),
                  jax.ShapeDtypeStruct((1024, CMAX), jnp.float32)],
        scratch_types=[pltpu.VMEM((TB,), jnp.float32),
                       pltpu.VMEM((CMAX + 16,), jnp.int32),
                       pltpu.VMEM((CMAX + 16,), jnp.float32),
                       pltpu.VMEM((L,), jnp.float32),
                       pltpu.VMEM((L,), jnp.int32)],
    )
    def kern(d2_hbm, t_hbm, cidx_hbm, cval_hbm, buf, cidx_v, cval_v, tvec_v,
             cnt_v):
        wid = lax.axis_index("s") * nc + lax.axis_index("c")
        base_q = wid * qw
        zero = jnp.zeros((L,), jnp.int32)
        iot = lax.iota(jnp.int32, L)
        for qi in range(qw):
            q = base_q + qi
            pltpu.sync_copy(t_hbm.at[q], tvec_v)

            for i in range((CMAX + 16) // L):
                cidx_v[pl.ds(i * L, L)] = zero
                cval_v[pl.ds(i * L, L)] = jnp.full((L,), PADV, jnp.float32)

            def chunk(c, carry):
                pltpu.sync_copy(d2_hbm.at[q, pl.ds(c * TB, TB)], buf)

                def vloop(v, carry):
                    off, colv = carry
                    val = buf[pl.ds(v * L, L)]
                    mask = val < tvec_v[...]
                    mi = jnp.where(mask, jnp.ones((L,), jnp.int32), zero)
                    cs = mi
                    for kk in (1, 2, 4, 8):
                        sh = jnp.maximum(iot - kk, 0)
                        g = cs.at[sh].get(mode="promise_in_bounds")
                        cs = cs + jnp.where(iot >= kk, g, zero)
                    tot = cs[L - 1]
                    lo = zero
                    tgt = iot + 1
                    for step in (8, 4, 2, 1):
                        probe = jnp.minimum(lo + (step - 1), L - 1)
                        pv = cs.at[probe].get(mode="promise_in_bounds")
                        lo = jnp.where(pv < tgt, lo + step, lo)
                    inv = jnp.minimum(lo, L - 1)
                    outv = val.at[inv].get(mode="promise_in_bounds")
                    outc = colv.at[inv].get(mode="promise_in_bounds")
                    o = jnp.minimum(off, CMAX)
                    cval_v[pl.ds(o, L)] = outv
                    cidx_v[pl.ds(o, L)] = outc
                    return off + tot, colv + L

                return lax.fori_loop(0, TB // L, vloop, carry)

            off, _ = lax.fori_loop(0, NBLK, chunk, (jnp.int32(0), iot))
            of = jnp.minimum(off, CMAX)
            cval_v[pl.ds(of, L)] = jnp.full((L,), PADV, jnp.float32)
            cidx_v[pl.ds(of, L)] = zero

            pltpu.sync_copy(cidx_v.at[pl.ds(0, CMAX)], cidx_hbm.at[q])
            pltpu.sync_copy(cval_v.at[pl.ds(0, CMAX)], cval_hbm.at[q])

    return kern(d2, threshb)


QT4 = 8


def _k4_body(cv_ref, ci_ref, out_ref):
    cv = cv_ref[...]
    ci = ci_ref[...]
    rank = jnp.zeros((QT4, CMAX), jnp.float32)
    for jb in range(CMAX // 128):
        vj = cv[:, jb * 128:(jb + 1) * 128]
        lt = (vj[:, None, :] < cv[:, :, None]).astype(jnp.float32)
        ig = lax.broadcasted_iota(jnp.int32, (QT4, CMAX, 128), 1)
        jg = lax.broadcasted_iota(jnp.int32, (QT4, CMAX, 128), 2) + jb * 128
        tie = jnp.logical_and(vj[:, None, :] == cv[:, :, None], jg < ig)
        rank += jnp.sum(lt + tie.astype(jnp.float32), axis=2)
    riota = (lax.broadcasted_iota(jnp.int32, (QT4, CMAX, NN), 2) + 1
             ).astype(jnp.float32)
    oh = (rank[:, :, None] == riota).astype(jnp.float32)
    out_ref[...] = jnp.sum(oh * ci[:, :, None], axis=1)


def _k4(cval, cidxf):
    return pl.pallas_call(
        _k4_body,
        grid=(1024 // QT4,),
        in_specs=[
            pl.BlockSpec((QT4, CMAX), lambda q: (q, 0)),
            pl.BlockSpec((QT4, CMAX), lambda q: (q, 0)),
        ],
        out_specs=pl.BlockSpec((QT4, NN), lambda q: (q, 0)),
        out_shape=jax.ShapeDtypeStruct((1024, NN), jnp.float32),
    )(cval, cidxf)


def _sqd(A, B):
    a2 = jnp.sum(A * A, axis=-1, keepdims=True)
    b2 = jnp.sum(B * B, axis=-1, keepdims=True)
    d2 = a2 + jnp.swapaxes(b2, -1, -2) - 2.0 * jnp.einsum('...md,...nd->...mn', A, B)
    return jnp.clip(d2, 0.0, None)


def _rbf(A, B, l, a):
    return a * jnp.exp(-_sqd(A, B) / (2.0 * l ** 2))


def kernel(x, trainX, trainy, l, a):
    a2 = jnp.sum(x * x, axis=-1, keepdims=True)
    b2 = jnp.sum(trainX * trainX, axis=-1, keepdims=True)
    tXp = jnp.pad(trainX, ((0, NPAD - N), (0, 0)))
    b2r = jnp.pad(b2, ((0, NPAD - N), (0, 0))).reshape(NBLK, 1, TB)

    d2, m, mx = _k1(x, tXp, a2, b2r)
    thresh = _find_thresholds(d2, m, mx)
    cidx, cval = _compact_sc(d2, thresh)
    idxf = _k4(cval, cidx.astype(jnp.float32))
    idx = idxf.astype(jnp.int32)

    ymean = 0.0
    nX = trainX[idx]
    ny = trainy[idx]
    ny = ny - ymean
    auto = _rbf(nX, nX, l, a)
    autoCov = jnp.linalg.inv(auto)
    crossCov = _rbf(x[:, None, :], nX, l, a)
    kWeights = crossCov @ autoCov
    y = kWeights @ ny
    yVar = a * jnp.ones(x.shape[0], dtype=x.dtype) - jnp.squeeze(
        kWeights @ jnp.swapaxes(crossCov, 1, 2))
    return (jnp.squeeze(y + ymean), yVar)
